# baseline (speedup 1.0000x reference)
import functools

import jax
import jax.numpy as jnp
from jax import lax
from jax.experimental import pallas as pl
from jax.experimental.pallas import tpu as pltpu
from jax.experimental.pallas import tpu_sc as plsc

N = 10000
E = 320000
H = 64
NWORK = 32
EPW = E // NWORK
SUBG = 40
GRP = 5
CH = SUBG * GRP
NCH = EPW // CH
SUBS = 80
ROWSG = EPW // SUBS
CHS = SUBS
NCHS = EPW // CHS
NP = 10240
NPS = NP // 16
W = 128
BE = 4000


def _mesh():
    return plsc.VectorSubcoreMesh(core_axis_name="c", subcore_axis_name="s")


def _sc_gather(tab, src4, dst4):
    @functools.partial(
        pl.kernel,
        mesh=_mesh(),
        out_type=[
            jax.ShapeDtypeStruct((E, W), jnp.float32),
            jax.ShapeDtypeStruct((E, W), jnp.float32),
        ],
        scratch_types=[
            pltpu.VMEM((GRP, SUBG), jnp.int32),
            pltpu.VMEM((GRP, SUBG), jnp.int32),
            pltpu.VMEM((GRP, SUBG), jnp.int32),
            pltpu.VMEM((GRP, SUBG), jnp.int32),
            pltpu.VMEM((CH, W), jnp.float32),
            pltpu.VMEM((CH, W), jnp.float32),
            pltpu.VMEM((CH, W), jnp.float32),
            pltpu.VMEM((CH, W), jnp.float32),
            pltpu.SemaphoreType.DMA,
        ],
    )
    def gk(tab_hbm, src_hbm, dst_hbm, hs_out, hd_out,
           s0, d0, s1, d1, rs0, rd0, rs1, rd1, sem):
        wid = lax.axis_index("c") * 16 + lax.axis_index("s")
        base = wid * EPW

        def load_idx(i, sbuf, dbuf):
            pltpu.sync_copy(src_hbm.at[wid, i], sbuf)
            pltpu.sync_copy(dst_hbm.at[wid, i], dbuf)

        def fire(sbuf, dbuf, rsbuf, rdbuf):
            for j in range(GRP):
                pltpu.async_copy(tab_hbm.at[sbuf.at[j]],
                                 rsbuf.at[pl.ds(j * SUBG, SUBG)], sem)
                pltpu.async_copy(tab_hbm.at[dbuf.at[j]],
                                 rdbuf.at[pl.ds(j * SUBG, SUBG)], sem)

        def drain_out(i, rsbuf, rdbuf):
            for j in range(GRP):
                pltpu.make_async_copy(tab_hbm.at[pl.ds(0, SUBG)],
                                      rsbuf.at[pl.ds(j * SUBG, SUBG)], sem).wait()
                pltpu.make_async_copy(tab_hbm.at[pl.ds(0, SUBG)],
                                      rdbuf.at[pl.ds(j * SUBG, SUBG)], sem).wait()
            pltpu.sync_copy(rsbuf, hs_out.at[pl.ds(base + i * CH, CH)])
            pltpu.sync_copy(rdbuf, hd_out.at[pl.ds(base + i * CH, CH)])

        load_idx(0, s0, d0)
        fire(s0, d0, rs0, rd0)

        def pair(t, carry):
            i = 2 * t
            load_idx(i + 1, s1, d1)
            fire(s1, d1, rs1, rd1)
            drain_out(i, rs0, rd0)
            load_idx(i + 2, s0, d0)
            fire(s0, d0, rs0, rd0)
            drain_out(i + 1, rs1, rd1)
            return carry

        lax.fori_loop(0, NCH // 2 - 1, pair, 0)
        load_idx(NCH - 1, s1, d1)
        fire(s1, d1, rs1, rd1)
        drain_out(NCH - 2, rs0, rd0)
        drain_out(NCH - 1, rs1, rd1)

    return gk(tab, src4, dst4)


def _sc_scatter(pm, dst3, zeros):
    @functools.partial(
        pl.kernel,
        mesh=_mesh(),
        out_type=jax.ShapeDtypeStruct((2, NP, W), jnp.float32),
        scratch_types=[
            pltpu.VMEM((ROWSG, SUBS), jnp.int32),
            pltpu.VMEM((CHS, W), jnp.float32),
            pltpu.VMEM((CHS, W), jnp.float32),
            pltpu.VMEM_SHARED((NP, W), jnp.float32),
            pltpu.SemaphoreType.DMA,
        ],
    )
    def sk(pm_hbm, dst_hbm, z_hbm, out_hbm, didx, buf0, buf1, acc, sem):
        c = lax.axis_index("c")
        s = lax.axis_index("s")
        wid = c * 16 + s
        pltpu.sync_copy(z_hbm.at[pl.ds(s * NPS, NPS)], acc.at[pl.ds(s * NPS, NPS)])
        pltpu.sync_copy(dst_hbm.at[wid], didx)
        plsc.subcore_barrier()
        base = wid * EPW

        def chunk(i):
            return pm_hbm.at[pl.ds(base + i * CHS, CHS)]

        pltpu.async_copy(chunk(0), buf0, sem)

        def body(t, carry):
            i = 2 * t
            pltpu.async_copy(chunk(i + 1), buf1, sem)
            pltpu.make_async_copy(chunk(i), buf0, sem).wait()
            pltpu.sync_copy(buf0, acc.at[didx.at[i]], add=True)
            pltpu.async_copy(chunk(i + 2), buf0, sem)
            pltpu.make_async_copy(chunk(i + 1), buf1, sem).wait()
            pltpu.sync_copy(buf1, acc.at[didx.at[i + 1]], add=True)
            return carry

        lax.fori_loop(0, (NCHS - 1) // 2, body, 0)
        pltpu.make_async_copy(chunk(NCHS - 1), buf0, sem).wait()
        pltpu.sync_copy(buf0, acc.at[didx.at[NCHS - 1]], add=True)
        plsc.subcore_barrier()
        pltpu.sync_copy(acc.at[pl.ds(s * NPS, NPS)],
                        out_hbm.at[c, pl.ds(s * NPS, NPS)])

    return sk(pm, dst3, zeros)


def _embed_body(nf_ref, w_ref, b_ref, h_ref, tab_ref):
    x = jnp.dot(nf_ref[...], w_ref[...], preferred_element_type=jnp.float32) + b_ref[...]
    h_ref[...] = x[:, :H]
    tab_ref[...] = x[:, H:H + W]


def _tc_embed(node_feats, wcat, bcat):
    bn = 2000
    return pl.pallas_call(
        _embed_body,
        grid=(N // bn,),
        in_specs=[
            pl.BlockSpec((bn, 128), lambda i: (i, 0)),
            pl.BlockSpec((128, H + W), lambda i: (0, 0)),
            pl.BlockSpec((1, H + W), lambda i: (0, 0)),
        ],
        out_specs=[
            pl.BlockSpec((bn, H), lambda i: (i, 0)),
            pl.BlockSpec((bn, W), lambda i: (i, 0)),
        ],
        out_shape=[
            jax.ShapeDtypeStruct((N, H), jnp.float32),
            jax.ShapeDtypeStruct((N, W), jnp.float32),
        ],
    )(node_feats, wcat, bcat)


def _edge_body(hs_ref, hd_ref, ef_ref, wf_ref, bf_ref, wab_ref, ba_ref, out_ref):
    m = jnp.maximum(
        hs_ref[:, :H]
        + jnp.dot(ef_ref[...], wf_ref[...], preferred_element_type=jnp.float32)
        + bf_ref[...], 0.0)
    q = jnp.sum(m * wab_ref[...], axis=1, keepdims=True)
    x = hd_ref[:, H:H + 1] + q + ba_ref[...]
    lg = jnp.where(x > 0, x, 0.01 * x)
    pe = jnp.exp(lg)
    out_ref[...] = jnp.concatenate(
        [m * pe, pe, jnp.zeros((m.shape[0], W - H - 1), jnp.float32)], axis=1)


def _tc_edge(hs, hd, ef, wf, bf, wab, ba):
    return pl.pallas_call(
        _edge_body,
        grid=(E // BE,),
        in_specs=[
            pl.BlockSpec((BE, W), lambda i: (i, 0)),
            pl.BlockSpec((BE, W), lambda i: (i, 0)),
            pl.BlockSpec((BE, 16), lambda i: (i, 0)),
            pl.BlockSpec((16, H), lambda i: (0, 0)),
            pl.BlockSpec((1, H), lambda i: (0, 0)),
            pl.BlockSpec((1, H), lambda i: (0, 0)),
            pl.BlockSpec((1, 1), lambda i: (0, 0)),
        ],
        out_specs=pl.BlockSpec((BE, W), lambda i: (i, 0)),
        out_shape=jax.ShapeDtypeStruct((E, W), jnp.float32),
    )(hs, hd, ef, wf, bf, wab, ba)


def _gru_block(ctx, h, wxzr, whzr, bzr, wxn, whn, bn):
    zr = ctx @ wxzr + h @ whzr + bzr
    zr = 1.0 / (1.0 + jnp.exp(-zr))
    z = zr[:, :H]
    r = zr[:, H:]
    n = jnp.tanh(ctx @ wxn + (r * h) @ whn + bn)
    return (1.0 - z) * n + z * h


def _ctx_from_c2(c2_ref):
    C = c2_ref[0, :N] + c2_ref[1, :N]
    ctx = C[:, :H] / (C[:, H:H + 1] + 1e-16)
    return jnp.where(ctx > 0, ctx, jnp.exp(jnp.minimum(ctx, 0.0)) - 1.0)


def _node_body(c2_ref, h_ref, wxzr_ref, whzr_ref, bzr_ref, wxn_ref, whn_ref,
               bn_ref, wnext_ref, h_out, tab_out):
    ctx = _ctx_from_c2(c2_ref)
    hn = _gru_block(ctx, h_ref[...], wxzr_ref[...], whzr_ref[...], bzr_ref[...],
                    wxn_ref[...], whn_ref[...], bn_ref[...])
    t = jnp.dot(hn, wnext_ref[...], preferred_element_type=jnp.float32)
    h_out[...] = hn
    tab_out[...] = t


def _tc_node(c2, h, wxzr, whzr, bzr, wxn, whn, bn, wnext):
    return pl.pallas_call(
        _node_body,
        out_shape=[
            jax.ShapeDtypeStruct((N, H), jnp.float32),
            jax.ShapeDtypeStruct((N, W), jnp.float32),
        ],
    )(c2, h, wxzr, whzr, bzr, wxn, whn, bn, wnext)


def _readout_body(c2_ref, h_ref, wxzr_ref, whzr_ref, bzr_ref, wxn_ref, whn_ref,
                  bn_ref, wgt_ref, wgb_ref, bg_ref, rwxzr_ref, rwhzr_ref,
                  rbzr_ref, rwxn_ref, rwhn_ref, rbn_ref, w1_ref, b1_ref,
                  w2_ref, b2_ref, out_ref):
    ctx = _ctx_from_c2(c2_ref)
    hn = _gru_block(ctx, h_ref[...], wxzr_ref[...], whzr_ref[...], bzr_ref[...],
                    wxn_ref[...], whn_ref[...], bn_ref[...])
    g = jnp.mean(hn, axis=0, keepdims=True)
    gt = jnp.sum(g * wgt_ref[...])
    gl = jnp.sum(hn * wgb_ref[...], axis=1, keepdims=True) + gt + bg_ref[...]
    gl = jnp.where(gl > 0, gl, 0.01 * gl)
    amax = jnp.max(gl)
    ae = jnp.exp(gl - amax)
    a = ae / jnp.sum(ae)
    ctxr = jnp.sum(a * hn, axis=0, keepdims=True)
    ctxr = jnp.where(ctxr > 0, ctxr, jnp.exp(jnp.minimum(ctxr, 0.0)) - 1.0)
    gn = _gru_block(ctxr, g, rwxzr_ref[...], rwhzr_ref[...], rbzr_ref[...],
                    rwxn_ref[...], rwhn_ref[...], rbn_ref[...])
    y = jnp.maximum(jnp.dot(gn, w1_ref[...], preferred_element_type=jnp.float32)
                    + b1_ref[...], 0.0)
    out_ref[...] = jnp.dot(y, w2_ref[...], preferred_element_type=jnp.float32) + b2_ref[...]


def _tc_readout(c2, h, args):
    return pl.pallas_call(
        _readout_body,
        out_shape=jax.ShapeDtypeStruct((1, 1), jnp.float32),
    )(c2, h, *args)


def kernel(graph, node_feats, edge_feats, params):
    p = params
    src = graph[0].astype(jnp.int32)
    dst = graph[1].astype(jnp.int32)
    src3g = src.reshape(NWORK, NCH, GRP, SUBG)
    dst3g = dst.reshape(NWORK, NCH, GRP, SUBG)
    dst3 = dst.reshape(NWORK, ROWSG, SUBS)

    Wm_top = p["W_msg"][:, :H, :]
    Wm_bot = p["W_msg"][:, H:, :]
    Wf = jnp.einsum("eh,lhk->lek", p["W_edge"], Wm_bot)
    bf = p["b_edge"] @ Wm_bot + p["b_msg"]
    Wa_top = p["W_att"][:, :H, 0]
    Wa_bot = p["W_att"][:, H:, 0]
    ba = p["b_att"][:, 0]

    def wtab(l):
        return jnp.concatenate(
            [Wm_top[l], Wa_top[l][:, None],
             jnp.zeros((H, W - H - 1), jnp.float32)], axis=1)

    wcat = jnp.concatenate([p["W_node"], p["W_node"] @ wtab(0)], axis=1)
    bcat = jnp.concatenate([p["b_node"], p["b_node"] @ wtab(0)], axis=0)[None, :]

    zeros = jnp.zeros((NP, W), jnp.float32)

    h, tab = _tc_embed(node_feats, wcat, bcat)

    L = p["W_msg"].shape[0]
    out = None
    for l in range(L):
        hs, hd = _sc_gather(tab, src3g, dst3g)
        pm = _tc_edge(hs, hd, edge_feats, Wf[l], bf[l][None, :],
                      Wa_bot[l][None, :], ba[l][None, None])
        c2 = _sc_scatter(pm, dst3, zeros)
        if l < L - 1:
            h, tab = _tc_node(
                c2, h, p["Wx_zr"][l], p["Wh_zr"][l], p["b_zr"][l][None, :],
                p["Wx_n"][l], p["Wh_n"][l], p["b_n"][l][None, :], wtab(l + 1))
        else:
            args = (
                p["Wx_zr"][l], p["Wh_zr"][l], p["b_zr"][l][None, :],
                p["Wx_n"][l], p["Wh_n"][l], p["b_n"][l][None, :],
                p["W_gatt"][:H, 0][None, :], p["W_gatt"][H:, 0][None, :],
                p["b_gatt"][None, :],
                p["rWx_zr"], p["rWh_zr"], p["rb_zr"][None, :],
                p["rWx_n"], p["rWh_n"], p["rb_n"][None, :],
                p["W1"], p["b1"][None, :], p["W2"], p["b2"][None, :],
            )
            out = _tc_readout(c2, h, args)
    return out.reshape((1,))


# speedup vs baseline: 1.0162x; 1.0162x over previous
import functools

import jax
import jax.numpy as jnp
from jax import lax
from jax.experimental import pallas as pl
from jax.experimental.pallas import tpu as pltpu
from jax.experimental.pallas import tpu_sc as plsc

N = 10000
E = 320000
H = 64
NWORK = 32
NSLAB = 2
ES = E // NSLAB
EPW = ES // NWORK
SUBG = 40
GRP = 5
CH = SUBG * GRP
NCH = EPW // CH
SUBS = 40
ROWSG = EPW // SUBS
CHS = SUBS
NCHS = EPW // CHS
NP = 10240
NPS = NP // 16
W = 128
BE = 4000


def _mesh():
    return plsc.VectorSubcoreMesh(core_axis_name="c", subcore_axis_name="s")


def _sc_gather(tab, src4, dst4):
    @functools.partial(
        pl.kernel,
        mesh=_mesh(),
        out_type=[
            jax.ShapeDtypeStruct((ES, W), jnp.float32),
            jax.ShapeDtypeStruct((ES, W), jnp.float32),
        ],
        scratch_types=[
            pltpu.VMEM((GRP, SUBG), jnp.int32),
            pltpu.VMEM((GRP, SUBG), jnp.int32),
            pltpu.VMEM((GRP, SUBG), jnp.int32),
            pltpu.VMEM((GRP, SUBG), jnp.int32),
            pltpu.VMEM((CH, W), jnp.float32),
            pltpu.VMEM((CH, W), jnp.float32),
            pltpu.VMEM((CH, W), jnp.float32),
            pltpu.VMEM((CH, W), jnp.float32),
            pltpu.SemaphoreType.DMA,
        ],
    )
    def gk(tab_hbm, src_hbm, dst_hbm, hs_out, hd_out,
           s0, d0, s1, d1, rs0, rd0, rs1, rd1, sem):
        wid = lax.axis_index("c") * 16 + lax.axis_index("s")
        base = wid * EPW

        def load_idx(i, sbuf, dbuf):
            pltpu.sync_copy(src_hbm.at[wid, i], sbuf)
            pltpu.sync_copy(dst_hbm.at[wid, i], dbuf)

        def fire(sbuf, dbuf, rsbuf, rdbuf):
            for j in range(GRP):
                pltpu.async_copy(tab_hbm.at[sbuf.at[j]],
                                 rsbuf.at[pl.ds(j * SUBG, SUBG)], sem)
                pltpu.async_copy(tab_hbm.at[dbuf.at[j]],
                                 rdbuf.at[pl.ds(j * SUBG, SUBG)], sem)

        def drain_out(i, rsbuf, rdbuf):
            for j in range(GRP):
                pltpu.make_async_copy(tab_hbm.at[pl.ds(0, SUBG)],
                                      rsbuf.at[pl.ds(j * SUBG, SUBG)], sem).wait()
                pltpu.make_async_copy(tab_hbm.at[pl.ds(0, SUBG)],
                                      rdbuf.at[pl.ds(j * SUBG, SUBG)], sem).wait()
            pltpu.sync_copy(rsbuf, hs_out.at[pl.ds(base + i * CH, CH)])
            pltpu.sync_copy(rdbuf, hd_out.at[pl.ds(base + i * CH, CH)])

        load_idx(0, s0, d0)
        fire(s0, d0, rs0, rd0)

        def pair(t, carry):
            i = 2 * t
            load_idx(i + 1, s1, d1)
            fire(s1, d1, rs1, rd1)
            drain_out(i, rs0, rd0)
            load_idx(i + 2, s0, d0)
            fire(s0, d0, rs0, rd0)
            drain_out(i + 1, rs1, rd1)
            return carry

        lax.fori_loop(0, (NCH - 1) // 2, pair, 0)
        drain_out(NCH - 1, rs0, rd0)

    return gk(tab, src4, dst4)


def _sc_scatter(pm, dst3, zeros):
    @functools.partial(
        pl.kernel,
        mesh=_mesh(),
        out_type=jax.ShapeDtypeStruct((2, NP, W), jnp.float32),
        scratch_types=[
            pltpu.VMEM((ROWSG, SUBS), jnp.int32),
            pltpu.VMEM((CHS, W), jnp.float32),
            pltpu.VMEM((CHS, W), jnp.float32),
            pltpu.VMEM_SHARED((NP, W), jnp.float32),
            pltpu.SemaphoreType.DMA,
        ],
    )
    def sk(pm_hbm, dst_hbm, z_hbm, out_hbm, didx, buf0, buf1, acc, sem):
        c = lax.axis_index("c")
        s = lax.axis_index("s")
        wid = c * 16 + s
        pltpu.sync_copy(z_hbm.at[pl.ds(s * NPS, NPS)], acc.at[pl.ds(s * NPS, NPS)])
        pltpu.sync_copy(dst_hbm.at[wid], didx)
        plsc.subcore_barrier()
        base = wid * EPW

        def chunk(i):
            return pm_hbm.at[pl.ds(base + i * CHS, CHS)]

        pltpu.async_copy(chunk(0), buf0, sem)

        def body(t, carry):
            i = 2 * t
            pltpu.async_copy(chunk(i + 1), buf1, sem)
            pltpu.make_async_copy(chunk(i), buf0, sem).wait()
            pltpu.sync_copy(buf0, acc.at[didx.at[i]], add=True)
            pltpu.async_copy(chunk(i + 2), buf0, sem)
            pltpu.make_async_copy(chunk(i + 1), buf1, sem).wait()
            pltpu.sync_copy(buf1, acc.at[didx.at[i + 1]], add=True)
            return carry

        lax.fori_loop(0, (NCHS - 1) // 2, body, 0)
        pltpu.make_async_copy(chunk(NCHS - 1), buf0, sem).wait()
        pltpu.sync_copy(buf0, acc.at[didx.at[NCHS - 1]], add=True)
        plsc.subcore_barrier()
        pltpu.sync_copy(acc.at[pl.ds(s * NPS, NPS)],
                        out_hbm.at[c, pl.ds(s * NPS, NPS)])

    return sk(pm, dst3, zeros)


def _embed_body(nf_ref, w_ref, b_ref, h_ref, tab_ref):
    x = jnp.dot(nf_ref[...], w_ref[...], preferred_element_type=jnp.float32) + b_ref[...]
    h_ref[...] = x[:, :H]
    tab_ref[...] = x[:, H:H + W]


def _tc_embed(node_feats, wcat, bcat):
    bn = 2000
    return pl.pallas_call(
        _embed_body,
        grid=(N // bn,),
        in_specs=[
            pl.BlockSpec((bn, 128), lambda i: (i, 0)),
            pl.BlockSpec((128, H + W), lambda i: (0, 0)),
            pl.BlockSpec((1, H + W), lambda i: (0, 0)),
        ],
        out_specs=[
            pl.BlockSpec((bn, H), lambda i: (i, 0)),
            pl.BlockSpec((bn, W), lambda i: (i, 0)),
        ],
        out_shape=[
            jax.ShapeDtypeStruct((N, H), jnp.float32),
            jax.ShapeDtypeStruct((N, W), jnp.float32),
        ],
    )(node_feats, wcat, bcat)


def _edge_body(hs_ref, hd_ref, ef_ref, wf_ref, bf_ref, wab_ref, ba_ref, out_ref):
    m = jnp.maximum(
        hs_ref[:, :H]
        + jnp.dot(ef_ref[...], wf_ref[...], preferred_element_type=jnp.float32)
        + bf_ref[...], 0.0)
    q = jnp.sum(m * wab_ref[...], axis=1, keepdims=True)
    x = hd_ref[:, H:H + 1] + q + ba_ref[...]
    lg = jnp.where(x > 0, x, 0.01 * x)
    pe = jnp.exp(lg)
    out_ref[...] = jnp.concatenate(
        [m * pe, pe, jnp.zeros((m.shape[0], W - H - 1), jnp.float32)], axis=1)


def _tc_edge(hs, hd, ef, wf, bf, wab, ba):
    return pl.pallas_call(
        _edge_body,
        grid=(ES // BE,),
        in_specs=[
            pl.BlockSpec((BE, W), lambda i: (i, 0)),
            pl.BlockSpec((BE, W), lambda i: (i, 0)),
            pl.BlockSpec((BE, 16), lambda i: (i, 0)),
            pl.BlockSpec((16, H), lambda i: (0, 0)),
            pl.BlockSpec((1, H), lambda i: (0, 0)),
            pl.BlockSpec((1, H), lambda i: (0, 0)),
            pl.BlockSpec((1, 1), lambda i: (0, 0)),
        ],
        out_specs=pl.BlockSpec((BE, W), lambda i: (i, 0)),
        out_shape=jax.ShapeDtypeStruct((ES, W), jnp.float32),
    )(hs, hd, ef, wf, bf, wab, ba)


def _gru_block(ctx, h, wxzr, whzr, bzr, wxn, whn, bn):
    zr = ctx @ wxzr + h @ whzr + bzr
    zr = 1.0 / (1.0 + jnp.exp(-zr))
    z = zr[:, :H]
    r = zr[:, H:]
    n = jnp.tanh(ctx @ wxn + (r * h) @ whn + bn)
    return (1.0 - z) * n + z * h


def _ctx_from_c2(c2a_ref, c2b_ref):
    C = c2a_ref[0, :N] + c2a_ref[1, :N] + c2b_ref[0, :N] + c2b_ref[1, :N]
    ctx = C[:, :H] / (C[:, H:H + 1] + 1e-16)
    return jnp.where(ctx > 0, ctx, jnp.exp(jnp.minimum(ctx, 0.0)) - 1.0)


def _node_body(c2a_ref, c2b_ref, h_ref, wxzr_ref, whzr_ref, bzr_ref, wxn_ref,
               whn_ref, bn_ref, wnext_ref, h_out, tab_out):
    ctx = _ctx_from_c2(c2a_ref, c2b_ref)
    hn = _gru_block(ctx, h_ref[...], wxzr_ref[...], whzr_ref[...], bzr_ref[...],
                    wxn_ref[...], whn_ref[...], bn_ref[...])
    t = jnp.dot(hn, wnext_ref[...], preferred_element_type=jnp.float32)
    h_out[...] = hn
    tab_out[...] = t


def _tc_node(c2a, c2b, h, wxzr, whzr, bzr, wxn, whn, bn, wnext):
    return pl.pallas_call(
        _node_body,
        out_shape=[
            jax.ShapeDtypeStruct((N, H), jnp.float32),
            jax.ShapeDtypeStruct((N, W), jnp.float32),
        ],
    )(c2a, c2b, h, wxzr, whzr, bzr, wxn, whn, bn, wnext)


def _readout_body(c2a_ref, c2b_ref, h_ref, wxzr_ref, whzr_ref, bzr_ref,
                  wxn_ref, whn_ref, bn_ref, wgt_ref, wgb_ref, bg_ref,
                  rwxzr_ref, rwhzr_ref, rbzr_ref, rwxn_ref, rwhn_ref, rbn_ref,
                  w1_ref, b1_ref, w2_ref, b2_ref, out_ref):
    ctx = _ctx_from_c2(c2a_ref, c2b_ref)
    hn = _gru_block(ctx, h_ref[...], wxzr_ref[...], whzr_ref[...], bzr_ref[...],
                    wxn_ref[...], whn_ref[...], bn_ref[...])
    g = jnp.mean(hn, axis=0, keepdims=True)
    gt = jnp.sum(g * wgt_ref[...])
    gl = jnp.sum(hn * wgb_ref[...], axis=1, keepdims=True) + gt + bg_ref[...]
    gl = jnp.where(gl > 0, gl, 0.01 * gl)
    amax = jnp.max(gl)
    ae = jnp.exp(gl - amax)
    a = ae / jnp.sum(ae)
    ctxr = jnp.sum(a * hn, axis=0, keepdims=True)
    ctxr = jnp.where(ctxr > 0, ctxr, jnp.exp(jnp.minimum(ctxr, 0.0)) - 1.0)
    gn = _gru_block(ctxr, g, rwxzr_ref[...], rwhzr_ref[...], rbzr_ref[...],
                    rwxn_ref[...], rwhn_ref[...], rbn_ref[...])
    y = jnp.maximum(jnp.dot(gn, w1_ref[...], preferred_element_type=jnp.float32)
                    + b1_ref[...], 0.0)
    out_ref[...] = jnp.dot(y, w2_ref[...], preferred_element_type=jnp.float32) + b2_ref[...]


def _tc_readout(c2a, c2b, h, args):
    return pl.pallas_call(
        _readout_body,
        out_shape=jax.ShapeDtypeStruct((1, 1), jnp.float32),
    )(c2a, c2b, h, *args)


def kernel(graph, node_feats, edge_feats, params):
    p = params
    src = graph[0].astype(jnp.int32)
    dst = graph[1].astype(jnp.int32)
    src4s, dst4s, dst3s, efs = [], [], [], []
    for s in range(NSLAB):
        sl = slice(s * ES, (s + 1) * ES)
        src4s.append(src[sl].reshape(NWORK, NCH, GRP, SUBG))
        dst4s.append(dst[sl].reshape(NWORK, NCH, GRP, SUBG))
        dst3s.append(dst[sl].reshape(NWORK, ROWSG, SUBS))
        efs.append(edge_feats[sl])

    Wm_top = p["W_msg"][:, :H, :]
    Wm_bot = p["W_msg"][:, H:, :]
    Wf = jnp.einsum("eh,lhk->lek", p["W_edge"], Wm_bot)
    bf = p["b_edge"] @ Wm_bot + p["b_msg"]
    Wa_top = p["W_att"][:, :H, 0]
    Wa_bot = p["W_att"][:, H:, 0]
    ba = p["b_att"][:, 0]

    def wtab(l):
        return jnp.concatenate(
            [Wm_top[l], Wa_top[l][:, None],
             jnp.zeros((H, W - H - 1), jnp.float32)], axis=1)

    wcat = jnp.concatenate([p["W_node"], p["W_node"] @ wtab(0)], axis=1)
    bcat = jnp.concatenate([p["b_node"], p["b_node"] @ wtab(0)], axis=0)[None, :]

    zeros = jnp.zeros((NP, W), jnp.float32)

    h, tab = _tc_embed(node_feats, wcat, bcat)

    L = p["W_msg"].shape[0]
    out = None
    for l in range(L):
        c2s = []
        for s in range(NSLAB):
            hs, hd = _sc_gather(tab, src4s[s], dst4s[s])
            pm = _tc_edge(hs, hd, efs[s], Wf[l], bf[l][None, :],
                          Wa_bot[l][None, :], ba[l][None, None])
            c2s.append(_sc_scatter(pm, dst3s[s], zeros))
        if l < L - 1:
            h, tab = _tc_node(
                c2s[0], c2s[1], h, p["Wx_zr"][l], p["Wh_zr"][l],
                p["b_zr"][l][None, :],
                p["Wx_n"][l], p["Wh_n"][l], p["b_n"][l][None, :], wtab(l + 1))
        else:
            args = (
                p["Wx_zr"][l], p["Wh_zr"][l], p["b_zr"][l][None, :],
                p["Wx_n"][l], p["Wh_n"][l], p["b_n"][l][None, :],
                p["W_gatt"][:H, 0][None, :], p["W_gatt"][H:, 0][None, :],
                p["b_gatt"][None, :],
                p["rWx_zr"], p["rWh_zr"], p["rb_zr"][None, :],
                p["rWx_n"], p["rWh_n"], p["rb_n"][None, :],
                p["W1"], p["b1"][None, :], p["W2"], p["b2"][None, :],
            )
            out = _tc_readout(c2s[0], c2s[1], h, args)
    return out.reshape((1,))
